# baseline (speedup 1.0000x reference)
import functools

import jax
import jax.numpy as jnp
from jax import lax
from jax.experimental import pallas as pl
from jax.experimental.pallas import tpu as pltpu
from jax.experimental.pallas import tpu_sc as plsc

B, F, NFEAT, NEMB = 16384, 26, 1000000, 16
NS, L = 16, 16
SPW = B // NS
CH = 32
NCHUNK = SPW // CH
CI = CH * F
NBLK = NFEAT // 128
REM = NFEAT - NBLK * 128
NROW = NFEAT // 8
BPW = (NBLK + NS - 1) // NS


@functools.partial(
    pl.kernel,
    out_type=(jax.ShapeDtypeStruct((B,), jnp.float32),
              jax.ShapeDtypeStruct((NROW, 8 * NEMB), jnp.float32)),
    mesh=plsc.VectorSubcoreMesh(core_axis_name="c", subcore_axis_name="s",
                                num_cores=1),
    compiler_params=pltpu.CompilerParams(needs_layout_passes=False),
    scratch_types=[
        pltpu.VMEM((NEMB, 128), jnp.float32),
        pltpu.VMEM((NEMB, 128), jnp.float32),
        pltpu.VMEM((CI,), jnp.int32),
        pltpu.VMEM((CI,), jnp.int32),
        pltpu.VMEM((CI + L,), jnp.int32),
        pltpu.VMEM((CI, 8 * NEMB), jnp.float32),
        pltpu.VMEM((CI + L,), jnp.float32),
        pltpu.VMEM((CH,), jnp.float32),
        pltpu.VMEM((4 * L,), jnp.float32),
        pltpu.SemaphoreType.DMA,
        pltpu.SemaphoreType.DMA,
    ],
)
def _fm_sc(x_hbm, embt_hbm, linw_hbm, wv_hbm, tail_hbm, out_hbm, tbl_hbm,
           tin_v, tout_v, idx_v, ridx_v, offs_v, rows_v, linv_v, out_v, wv_v,
           sem_e, sem_l):
    sid = lax.axis_index("s")
    lanes = lax.iota(jnp.int32, L)

    def tr_block(tj, carry):
        @pl.when(tj < NBLK)
        def _do():
            pltpu.sync_copy(embt_hbm.at[:, pl.ds(tj * 128, 128)], tin_v)
            for r in range(16):
                for k in range(8):
                    col = r * 8 + k
                    v = plsc.load_gather(
                        tin_v, [lanes, jnp.full((L,), col, jnp.int32)])
                    tout_v[r, pl.ds(k * NEMB, NEMB)] = v
            pltpu.sync_copy(tout_v, tbl_hbm.at[pl.ds(tj * 16, 16), :])
        return carry

    lax.fori_loop(0, BPW, lambda i, c: tr_block(sid * BPW + i, c), 0)

    @pl.when(sid == NS - 1)
    def _tail():
        pltpu.sync_copy(tail_hbm, tout_v.at[pl.ds(0, 8), :])
        pltpu.sync_copy(tout_v.at[pl.ds(0, 8), :],
                        tbl_hbm.at[pl.ds(NBLK * 16, 8), :])

    plsc.subcore_barrier()

    pltpu.sync_copy(wv_hbm, wv_v)
    wvec = wv_v[pl.ds(0, L)]
    wsum_vec = wv_v[pl.ds(L, L)]
    cvec = wv_v[pl.ds(2 * L, L)]
    tail_mask = lanes < (F - L)
    base = sid * SPW

    def chunk_body(c, carry):
        cb = base + c * CH
        pltpu.sync_copy(x_hbm.at[pl.ds(cb * F, CI)], idx_v)

        def split_body(i, carry2):
            iv = idx_v[pl.ds(i * L, L)]
            ridx_v[pl.ds(i * L, L)] = jnp.right_shift(iv, 3)
            offs_v[pl.ds(i * L, L)] = (iv & 7) * NEMB
            return carry2

        lax.fori_loop(0, CI // L, split_body, 0)
        cp_e = pltpu.async_copy(tbl_hbm.at[ridx_v], rows_v, sem_e)
        cp_l = pltpu.async_copy(linw_hbm.at[idx_v],
                                linv_v.at[pl.ds(0, CI)], sem_l)
        cp_e.wait()
        cp_l.wait()

        def group_body(g, carry2):
            acc = jnp.zeros((L,), jnp.float32)
            for j in range(L):
                rb = (g * L + j) * F
                oa = offs_v[pl.ds(rb, L)]
                ob = offs_v[pl.ds(rb + L, L)]
                S = jnp.zeros((L,), jnp.float32)
                Q = S
                for f in range(F):
                    o = oa[f] if f < L else ob[f - L]
                    v = rows_v[rb + f, pl.ds(o, L)]
                    S = S + v
                    Q = Q + v * v
                fm = 0.5 * (S * S - Q)
                la = linv_v[pl.ds(rb, L)]
                lb = jnp.where(tail_mask, linv_v[pl.ds(rb + L, L)], 0.0)
                t = fm * wvec + (la + lb) * wsum_vec + cvec
                acc = jnp.where(lanes == j, jnp.sum(t), acc)
            out_v[pl.ds(g * L, L)] = acc
            return carry2

        lax.fori_loop(0, CH // L, group_body, 0)
        pltpu.sync_copy(out_v, out_hbm.at[pl.ds(cb, CH)])
        return carry

    lax.fori_loop(0, NCHUNK, chunk_body, 0)


def kernel(x, emb_table, lin_w, lin_bias, clf_W, clf_b):
    wvec = clf_W[:, 0].astype(jnp.float32)
    wsum = jnp.sum(wvec)
    const = lin_bias * wsum + clf_b[0]
    wv = jnp.concatenate([
        wvec,
        jnp.full((L,), 1.0, jnp.float32) * wsum,
        jnp.full((L,), 1.0, jnp.float32) * (const / L),
        jnp.zeros((L,), jnp.float32),
    ])
    tail_rm = emb_table[NBLK * 128:, :].reshape(8, 128)
    out, _ = _fm_sc(x.reshape(-1), emb_table.T, lin_w, wv, tail_rm)
    return out.reshape(B, 1)


# speedup vs baseline: 1.7216x; 1.7216x over previous
import functools

import jax
import jax.numpy as jnp
from jax import lax
from jax.experimental import pallas as pl
from jax.experimental.pallas import tpu as pltpu
from jax.experimental.pallas import tpu_sc as plsc

B, F, NFEAT, NEMB = 16384, 26, 1000000, 16
NC, NS, L = 2, 16, 16
NW = NC * NS
SPW = B // NW
CH = 128
NCHUNK = SPW // CH
CI = CH * F

GC = 1024
NG = 976
GPT = NG // NS
REMC = 999936 - NG * GC
NROW = NFEAT // 8


@functools.partial(
    pl.kernel,
    out_type=jax.ShapeDtypeStruct((NROW, 8 * NEMB), jnp.float32),
    mesh=plsc.VectorSubcoreMesh(core_axis_name="c", subcore_axis_name="s",
                                num_cores=1),
    compiler_params=pltpu.CompilerParams(needs_layout_passes=False),
    scratch_types=[
        pltpu.VMEM((NEMB, GC), jnp.float32),
        pltpu.VMEM((NEMB, GC), jnp.float32),
        pltpu.VMEM((GC // 8, 128), jnp.float32),
        pltpu.VMEM((GC // 8, 128), jnp.float32),
        pltpu.SemaphoreType.DMA,
        pltpu.SemaphoreType.DMA,
        pltpu.SemaphoreType.DMA,
        pltpu.SemaphoreType.DMA,
    ],
)
def _tr_sc(embt_hbm, tail_hbm, tbl_hbm,
           tin_a, tin_b, tout_a, tout_b, isem_a, isem_b, osem_a, osem_b):
    sid = lax.axis_index("s")
    lanes = lax.iota(jnp.int32, L)
    g0 = sid * GPT

    def start_in(g, buf, sem):
        return pltpu.async_copy(
            embt_hbm.at[:, pl.ds((g0 + g) * GC, GC)], buf, sem)

    def transpose_buf(tin, tout):
        def row_body(r, carry):
            for k in range(8):
                c = r * 8 + k
                v = plsc.load_gather(
                    tin, [lanes, jnp.full((L,), 1, jnp.int32) * c])
                tout[r, pl.ds(k * NEMB, NEMB)] = v
            return carry
        lax.fori_loop(0, GC // 8, row_body, 0)

    def start_out(g, buf, sem):
        return pltpu.async_copy(
            buf, tbl_hbm.at[pl.ds((g0 + g) * (GC // 8), GC // 8), :], sem)

    start_in(0, tin_a, isem_a)

    def pair_body(i, carry):
        start_in(2 * i + 1, tin_b, isem_b)
        pltpu.make_async_copy(
            embt_hbm.at[:, pl.ds(0, GC)], tin_a, isem_a).wait()

        @pl.when(i > 0)
        def _drain_a():
            pltpu.make_async_copy(
                tout_a, tbl_hbm.at[pl.ds(0, GC // 8), :], osem_a).wait()

        transpose_buf(tin_a, tout_a)
        start_out(2 * i, tout_a, osem_a)

        @pl.when(i < 29)
        def _next_a():
            start_in(2 * i + 2, tin_a, isem_a)

        pltpu.make_async_copy(
            embt_hbm.at[:, pl.ds(0, GC)], tin_b, isem_b).wait()

        @pl.when(i > 0)
        def _drain_b():
            pltpu.make_async_copy(
                tout_b, tbl_hbm.at[pl.ds(0, GC // 8), :], osem_b).wait()

        transpose_buf(tin_b, tout_b)
        start_out(2 * i + 1, tout_b, osem_b)
        return carry

    lax.fori_loop(0, GPT // 2, pair_body, 0)

    start_in(GPT - 1, tin_a, isem_a)
    pltpu.make_async_copy(embt_hbm.at[:, pl.ds(0, GC)], tin_a, isem_a).wait()
    pltpu.make_async_copy(
        tout_a, tbl_hbm.at[pl.ds(0, GC // 8), :], osem_a).wait()
    transpose_buf(tin_a, tout_a)
    start_out(GPT - 1, tout_a, osem_a)
    pltpu.make_async_copy(
        tout_a, tbl_hbm.at[pl.ds(0, GC // 8), :], osem_a).wait()
    pltpu.make_async_copy(
        tout_b, tbl_hbm.at[pl.ds(0, GC // 8), :], osem_b).wait()

    @pl.when(sid == 0)
    def _rem():
        pltpu.sync_copy(embt_hbm.at[:, pl.ds(NG * GC, REMC)],
                        tin_a.at[:, pl.ds(0, REMC)])

        def row_body(r, carry):
            for k in range(8):
                c = r * 8 + k
                v = plsc.load_gather(
                    tin_a, [lanes, jnp.full((L,), 1, jnp.int32) * c])
                tout_a[r, pl.ds(k * NEMB, NEMB)] = v
            return carry

        lax.fori_loop(0, REMC // 8, row_body, 0)
        pltpu.sync_copy(tout_a.at[pl.ds(0, REMC // 8), :],
                        tbl_hbm.at[pl.ds(NG * GC // 8, REMC // 8), :])
        pltpu.sync_copy(tail_hbm, tout_b.at[pl.ds(0, 8), :])
        pltpu.sync_copy(tout_b.at[pl.ds(0, 8), :],
                        tbl_hbm.at[pl.ds((NG * GC + REMC) // 8, 8), :])


@functools.partial(
    pl.kernel,
    out_type=jax.ShapeDtypeStruct((B,), jnp.float32),
    mesh=plsc.VectorSubcoreMesh(core_axis_name="c", subcore_axis_name="s"),
    compiler_params=pltpu.CompilerParams(
        needs_layout_passes=False, use_tc_tiling_on_sc=False),
    scratch_types=[
        pltpu.VMEM((CI,), jnp.int32),
        pltpu.VMEM((CI, L), jnp.float32),
        pltpu.VMEM((CI + L,), jnp.float32),
        pltpu.VMEM((CH,), jnp.float32),
        pltpu.VMEM((4 * L,), jnp.float32),
        pltpu.SemaphoreType.DMA,
        pltpu.SemaphoreType.DMA,
    ],
)
def _fm_sc(x_hbm, emb_hbm, linw_hbm, wv_hbm, out_hbm,
           idx_v, rows_v, linv_v, out_v, wv_v, sem_e, sem_l):
    wid = lax.axis_index("s") * NC + lax.axis_index("c")
    base = wid * SPW
    pltpu.sync_copy(wv_hbm, wv_v)
    wvec = wv_v[pl.ds(0, L)]
    wsum_vec = wv_v[pl.ds(L, L)]
    cvec = wv_v[pl.ds(2 * L, L)]
    lanes = lax.iota(jnp.int32, L)
    tail_mask = lanes < (F - L)

    def chunk_body(c, carry):
        cb = base + c * CH
        pltpu.sync_copy(x_hbm.at[pl.ds(cb * F, CI)], idx_v)
        cp_e = pltpu.async_copy(emb_hbm.at[idx_v], rows_v, sem_e)
        cp_l = pltpu.async_copy(linw_hbm.at[idx_v],
                                linv_v.at[pl.ds(0, CI)], sem_l)
        cp_e.wait()
        cp_l.wait()

        def group_body(g, carry2):
            acc = jnp.zeros((L,), jnp.float32)
            for j in range(L):
                rb = (g * L + j) * F
                v0 = rows_v[rb, :]
                S = v0
                Q = v0 * v0
                for f in range(1, F):
                    v = rows_v[rb + f, :]
                    S = S + v
                    Q = Q + v * v
                fm = 0.5 * (S * S - Q)
                la = linv_v[pl.ds(rb, L)]
                lb = jnp.where(tail_mask, linv_v[pl.ds(rb + L, L)], 0.0)
                t = fm * wvec + (la + lb) * wsum_vec + cvec
                acc = jnp.where(lanes == j, jnp.sum(t), acc)
            out_v[pl.ds(g * L, L)] = acc
            return carry2

        lax.fori_loop(0, CH // L, group_body, 0)
        pltpu.sync_copy(out_v, out_hbm.at[pl.ds(cb, CH)])
        return carry

    lax.fori_loop(0, NCHUNK, chunk_body, 0)


def kernel(x, emb_table, lin_w, lin_bias, clf_W, clf_b):
    wvec = clf_W[:, 0].astype(jnp.float32)
    wsum = jnp.sum(wvec)
    const = lin_bias * wsum + clf_b[0]
    wv = jnp.concatenate([
        wvec,
        jnp.full((L,), 1.0, jnp.float32) * wsum,
        jnp.full((L,), 1.0, jnp.float32) * (const / L),
        jnp.zeros((L,), jnp.float32),
    ])
    tail_rm = emb_table[NFEAT - 64:, :].reshape(8, 128)
    tbl = _tr_sc(emb_table.T, tail_rm)
    out = _fm_sc(x.reshape(-1), tbl.reshape(NFEAT, NEMB), lin_w, wv)
    return out.reshape(B, 1)


# speedup vs baseline: 4.1048x; 2.3843x over previous
import functools

import jax
import jax.numpy as jnp
from jax import lax
from jax.experimental import pallas as pl
from jax.experimental.pallas import tpu as pltpu
from jax.experimental.pallas import tpu_sc as plsc

B, F, NFEAT, NEMB = 16384, 26, 1000000, 16
NC, NS, L = 2, 16, 16
NW = NC * NS
SPW = B // NW
CH = 128
NCHUNK = SPW // CH
CI = CH * F

NROW = NFEAT // 8
TBW = 8192
TGRID = (NFEAT + TBW - 1) // TBW


def _tr_tc_body(in_ref, out_ref):
    for k in range(8):
        out_ref[:, k * NEMB:(k + 1) * NEMB] = (
            in_ref[:, k * (TBW // 8):(k + 1) * (TBW // 8)].T)


_tr_tc = pl.pallas_call(
    _tr_tc_body,
    grid=(TGRID,),
    in_specs=[pl.BlockSpec((NEMB, TBW), lambda i: (0, i))],
    out_specs=pl.BlockSpec((TBW // 8, 8 * NEMB), lambda i: (i, 0)),
    out_shape=jax.ShapeDtypeStruct((NROW, 8 * NEMB), jnp.float32),
)


@functools.partial(
    pl.kernel,
    out_type=jax.ShapeDtypeStruct((B,), jnp.float32),
    mesh=plsc.VectorSubcoreMesh(core_axis_name="c", subcore_axis_name="s"),
    compiler_params=pltpu.CompilerParams(
        needs_layout_passes=False, use_tc_tiling_on_sc=False),
    scratch_types=[
        pltpu.VMEM((CI,), jnp.int32),
        pltpu.VMEM((CI,), jnp.int32),
        pltpu.VMEM((CI, L), jnp.float32),
        pltpu.VMEM((CI + L,), jnp.float32),
        pltpu.VMEM((CH,), jnp.float32),
        pltpu.VMEM((4 * L,), jnp.float32),
        pltpu.SemaphoreType.DMA,
        pltpu.SemaphoreType.DMA,
    ],
)
def _fm_sc(x_hbm, emb_hbm, linw_hbm, wv_hbm, out_hbm,
           idx_v, sidx_v, rows_v, linv_v, out_v, wv_v, sem_e, sem_l):
    wid = lax.axis_index("s") * NC + lax.axis_index("c")
    base = wid * SPW
    pltpu.sync_copy(wv_hbm, wv_v)
    wvec = wv_v[pl.ds(0, L)]
    wsum_vec = wv_v[pl.ds(L, L)]
    cvec = wv_v[pl.ds(2 * L, L)]
    lanes = lax.iota(jnp.int32, L)
    tail_mask = lanes < (F - L)

    def chunk_body(c, carry):
        cb = base + c * CH
        pltpu.sync_copy(x_hbm.at[pl.ds(cb * F, CI)], idx_v)

        def sig_body(i, carry2):
            iv = idx_v[pl.ds(i * L, L)]
            sidx_v[pl.ds(i * L, L)] = (
                (iv & -8192) | ((iv & 1023) << 3) | ((iv >> 10) & 7))
            return carry2

        lax.fori_loop(0, CI // L, sig_body, 0)
        cp_e = pltpu.async_copy(emb_hbm.at[sidx_v], rows_v, sem_e)
        cp_l = pltpu.async_copy(linw_hbm.at[idx_v],
                                linv_v.at[pl.ds(0, CI)], sem_l)
        cp_e.wait()
        cp_l.wait()

        def group_body(g, carry2):
            acc = jnp.zeros((L,), jnp.float32)
            for j in range(L):
                rb = (g * L + j) * F
                v0 = rows_v[rb, :]
                S = v0
                Q = v0 * v0
                for f in range(1, F):
                    v = rows_v[rb + f, :]
                    S = S + v
                    Q = Q + v * v
                fm = 0.5 * (S * S - Q)
                la = linv_v[pl.ds(rb, L)]
                lb = jnp.where(tail_mask, linv_v[pl.ds(rb + L, L)], 0.0)
                t = fm * wvec + (la + lb) * wsum_vec + cvec
                acc = jnp.where(lanes == j, jnp.sum(t), acc)
            out_v[pl.ds(g * L, L)] = acc
            return carry2

        lax.fori_loop(0, CH // L, group_body, 0)
        pltpu.sync_copy(out_v, out_hbm.at[pl.ds(cb, CH)])
        return carry

    lax.fori_loop(0, NCHUNK, chunk_body, 0)


def kernel(x, emb_table, lin_w, lin_bias, clf_W, clf_b):
    wvec = clf_W[:, 0].astype(jnp.float32)
    wsum = jnp.sum(wvec)
    const = lin_bias * wsum + clf_b[0]
    wv = jnp.concatenate([
        wvec,
        jnp.full((L,), 1.0, jnp.float32) * wsum,
        jnp.full((L,), 1.0, jnp.float32) * (const / L),
        jnp.zeros((L,), jnp.float32),
    ])
    tbl = _tr_tc(emb_table.T)
    out = _fm_sc(x.reshape(-1), tbl.reshape(NFEAT, NEMB), lin_w, wv)
    return out.reshape(B, 1)


# speedup vs baseline: 4.1064x; 1.0004x over previous
import functools

import jax
import jax.numpy as jnp
from jax import lax
from jax.experimental import pallas as pl
from jax.experimental.pallas import tpu as pltpu
from jax.experimental.pallas import tpu_sc as plsc

B, F, NFEAT, NEMB = 16384, 26, 1000000, 16
NC, NS, L = 2, 16, 16
NW = NC * NS
SPW = B // NW
CH = 128
NCHUNK = SPW // CH
CI = CH * F

TBW = 8192
TGRID = (NFEAT + TBW - 1) // TBW
NROW = TGRID * TBW // 8


def _tr_tc_body(in_ref, out_ref):
    for k in range(8):
        out_ref[:, k * NEMB:(k + 1) * NEMB] = (
            in_ref[:, k * (TBW // 8):(k + 1) * (TBW // 8)].T)


_tr_tc = pl.pallas_call(
    _tr_tc_body,
    grid=(TGRID,),
    in_specs=[pl.BlockSpec((NEMB, TBW), lambda i: (0, i))],
    out_specs=pl.BlockSpec((TBW // 8, 8 * NEMB), lambda i: (i, 0)),
    out_shape=jax.ShapeDtypeStruct((NROW, 8 * NEMB), jnp.float32),
)


@functools.partial(
    pl.kernel,
    out_type=jax.ShapeDtypeStruct((B,), jnp.float32),
    mesh=plsc.VectorSubcoreMesh(core_axis_name="c", subcore_axis_name="s"),
    compiler_params=pltpu.CompilerParams(
        needs_layout_passes=False, use_tc_tiling_on_sc=False),
    scratch_types=[
        pltpu.VMEM((CI,), jnp.int32),
        pltpu.VMEM((CI,), jnp.int32),
        pltpu.VMEM((CI, L), jnp.float32),
        pltpu.VMEM((CI + L,), jnp.float32),
        pltpu.VMEM((CH,), jnp.float32),
        pltpu.VMEM((4 * L,), jnp.float32),
        pltpu.SemaphoreType.DMA,
        pltpu.SemaphoreType.DMA,
    ],
)
def _fm_sc(x_hbm, emb_hbm, linw_hbm, wv_hbm, out_hbm,
           idx_v, sidx_v, rows_v, linv_v, out_v, wv_v, sem_e, sem_l):
    wid = lax.axis_index("s") * NC + lax.axis_index("c")
    base = wid * SPW
    pltpu.sync_copy(wv_hbm, wv_v)
    wvec = wv_v[pl.ds(0, L)]
    wsum_vec = wv_v[pl.ds(L, L)]
    cvec = wv_v[pl.ds(2 * L, L)]
    lanes = lax.iota(jnp.int32, L)
    tail_mask = lanes < (F - L)

    def chunk_body(c, carry):
        cb = base + c * CH
        pltpu.sync_copy(x_hbm.at[pl.ds(cb * F, CI)], idx_v)

        def sig_body(i, carry2):
            iv = idx_v[pl.ds(i * L, L)]
            sidx_v[pl.ds(i * L, L)] = (
                (iv & -8192) | ((iv & 1023) << 3) | ((iv >> 10) & 7))
            return carry2

        lax.fori_loop(0, CI // L, sig_body, 0)
        cp_e = pltpu.async_copy(emb_hbm.at[sidx_v], rows_v, sem_e)
        cp_l = pltpu.async_copy(linw_hbm.at[idx_v],
                                linv_v.at[pl.ds(0, CI)], sem_l)
        cp_e.wait()
        cp_l.wait()

        def group_body(g, carry2):
            acc = jnp.zeros((L,), jnp.float32)
            for j in range(L):
                rb = (g * L + j) * F
                v0 = rows_v[rb, :]
                S = v0
                Q = v0 * v0
                for f in range(1, F):
                    v = rows_v[rb + f, :]
                    S = S + v
                    Q = Q + v * v
                fm = 0.5 * (S * S - Q)
                la = linv_v[pl.ds(rb, L)]
                lb = jnp.where(tail_mask, linv_v[pl.ds(rb + L, L)], 0.0)
                t = fm * wvec + (la + lb) * wsum_vec + cvec
                acc = jnp.where(lanes == j, jnp.sum(t), acc)
            out_v[pl.ds(g * L, L)] = acc
            return carry2

        lax.fori_loop(0, CH // L, group_body, 0)
        pltpu.sync_copy(out_v, out_hbm.at[pl.ds(cb, CH)])
        return carry

    lax.fori_loop(0, NCHUNK, chunk_body, 0)


def kernel(x, emb_table, lin_w, lin_bias, clf_W, clf_b):
    wvec = clf_W[:, 0].astype(jnp.float32)
    wsum = jnp.sum(wvec)
    const = lin_bias * wsum + clf_b[0]
    wv = jnp.concatenate([
        wvec,
        jnp.full((L,), 1.0, jnp.float32) * wsum,
        jnp.full((L,), 1.0, jnp.float32) * (const / L),
        jnp.zeros((L,), jnp.float32),
    ])
    tbl = _tr_tc(emb_table.T)
    out = _fm_sc(x.reshape(-1), tbl.reshape(NROW * 8, NEMB), lin_w, wv)
    return out.reshape(B, 1)


# speedup vs baseline: 4.1239x; 1.0043x over previous
import functools

import jax
import jax.numpy as jnp
from jax import lax
from jax.experimental import pallas as pl
from jax.experimental.pallas import tpu as pltpu
from jax.experimental.pallas import tpu_sc as plsc

B, F, NFEAT, NEMB = 16384, 26, 1000000, 16
NC, NS, L = 2, 16, 16
NW = NC * NS
SPW = B // NW
CH = 128
NCHUNK = SPW // CH
CI = CH * F

TBW = 8192
TGRID = (NFEAT + TBW - 1) // TBW
NROW = TGRID * TBW // 8


def _tr_tc_body(in_ref, out_ref):
    w = in_ref[...].T
    for k in range(8):
        out_ref[:, k * NEMB:(k + 1) * NEMB] = (
            w[k * (TBW // 8):(k + 1) * (TBW // 8), :])


_tr_tc = pl.pallas_call(
    _tr_tc_body,
    grid=(TGRID,),
    in_specs=[pl.BlockSpec((NEMB, TBW), lambda i: (0, i))],
    out_specs=pl.BlockSpec((TBW // 8, 8 * NEMB), lambda i: (i, 0)),
    out_shape=jax.ShapeDtypeStruct((NROW, 8 * NEMB), jnp.float32),
)


@functools.partial(
    pl.kernel,
    out_type=jax.ShapeDtypeStruct((B,), jnp.float32),
    mesh=plsc.VectorSubcoreMesh(core_axis_name="c", subcore_axis_name="s"),
    compiler_params=pltpu.CompilerParams(
        needs_layout_passes=False, use_tc_tiling_on_sc=False),
    scratch_types=[
        pltpu.VMEM((CI,), jnp.int32),
        pltpu.VMEM((CI,), jnp.int32),
        pltpu.VMEM((CI, L), jnp.float32),
        pltpu.VMEM((CI + L,), jnp.float32),
        pltpu.VMEM((CH,), jnp.float32),
        pltpu.VMEM((4 * L,), jnp.float32),
        pltpu.SemaphoreType.DMA,
        pltpu.SemaphoreType.DMA,
    ],
)
def _fm_sc(x_hbm, emb_hbm, linw_hbm, wv_hbm, out_hbm,
           idx_v, sidx_v, rows_v, linv_v, out_v, wv_v, sem_e, sem_l):
    wid = lax.axis_index("s") * NC + lax.axis_index("c")
    base = wid * SPW
    pltpu.sync_copy(wv_hbm, wv_v)
    wvec = wv_v[pl.ds(0, L)]
    wsum_vec = wv_v[pl.ds(L, L)]
    cvec = wv_v[pl.ds(2 * L, L)]
    lanes = lax.iota(jnp.int32, L)
    tail_mask = lanes < (F - L)

    def chunk_body(c, carry):
        cb = base + c * CH
        pltpu.sync_copy(x_hbm.at[pl.ds(cb * F, CI)], idx_v)

        def sig_body(i, carry2):
            iv = idx_v[pl.ds(i * L, L)]
            sidx_v[pl.ds(i * L, L)] = (
                (iv & -8192) | ((iv & 1023) << 3) | ((iv >> 10) & 7))
            return carry2

        lax.fori_loop(0, CI // L, sig_body, 0)
        cp_e = pltpu.async_copy(emb_hbm.at[sidx_v], rows_v, sem_e)
        cp_l = pltpu.async_copy(linw_hbm.at[idx_v],
                                linv_v.at[pl.ds(0, CI)], sem_l)
        cp_e.wait()
        cp_l.wait()

        def group_body(g, carry2):
            acc = jnp.zeros((L,), jnp.float32)
            for j in range(L):
                rb = (g * L + j) * F
                v0 = rows_v[rb, :]
                S = v0
                Q = v0 * v0
                for f in range(1, F):
                    v = rows_v[rb + f, :]
                    S = S + v
                    Q = Q + v * v
                fm = 0.5 * (S * S - Q)
                la = linv_v[pl.ds(rb, L)]
                lb = jnp.where(tail_mask, linv_v[pl.ds(rb + L, L)], 0.0)
                t = fm * wvec + (la + lb) * wsum_vec + cvec
                acc = jnp.where(lanes == j, jnp.sum(t), acc)
            out_v[pl.ds(g * L, L)] = acc
            return carry2

        lax.fori_loop(0, CH // L, group_body, 0)
        pltpu.sync_copy(out_v, out_hbm.at[pl.ds(cb, CH)])
        return carry

    lax.fori_loop(0, NCHUNK, chunk_body, 0)


def kernel(x, emb_table, lin_w, lin_bias, clf_W, clf_b):
    wvec = clf_W[:, 0].astype(jnp.float32)
    wsum = jnp.sum(wvec)
    const = lin_bias * wsum + clf_b[0]
    wv = jnp.concatenate([
        wvec,
        jnp.full((L,), 1.0, jnp.float32) * wsum,
        jnp.full((L,), 1.0, jnp.float32) * (const / L),
        jnp.zeros((L,), jnp.float32),
    ])
    tbl = _tr_tc(emb_table.T)
    out = _fm_sc(x.reshape(-1), tbl.reshape(NROW * 8, NEMB), lin_w, wv)
    return out.reshape(B, 1)
